# baseline (speedup 1.0000x reference)
import functools

import jax
import jax.numpy as jnp
from jax import lax
from jax.experimental import pallas as pl
from jax.experimental.pallas import tpu as pltpu
from jax.experimental.pallas import tpu_sc as plsc

NC = 2
NS = 16
L = 16
NW = NC * NS
E = 320000
H = 128
N0 = 10000
CE = 80
PERW = E // NW
NCHUNK = PERW // CE
SW = 32
CEG = 40

DC = 640
SEG = 10240
NSEG = NW * SEG

NP1, NP2, NP3, NP4 = 10240, 5120, 2560, 1280

_mesh = plsc.VectorSubcoreMesh(core_axis_name="c", subcore_axis_name="s")
_SC_PARAMS = pltpu.CompilerParams(needs_layout_passes=False)


def _zero1d(ref, m):
    z = jnp.zeros((L,), ref.dtype)

    def body(i, _):
        ref[pl.ds(i * L, L)] = z
        return 0

    lax.fori_loop(0, m // L, body, 0)


def _zero_shared_1d(s, acc_sh, zbuf, m):
    _zero1d(zbuf, CE)
    chunk = m // NS

    def body(i, _):
        pltpu.sync_copy(zbuf, acc_sh.at[pl.ds(s * chunk + i * CE, CE)])
        return 0

    lax.fori_loop(0, chunk // CE, body, 0)


def _deg_body(npad, dst_hbm, out_hbm, acc_sh, onesbuf, dstbuf, stage):
    c = lax.axis_index("c")
    s = lax.axis_index("s")
    w = s * NC + c
    chunk = npad // NS
    _zero_shared_1d(s, acc_sh, onesbuf, npad)
    one = jnp.ones((L,), jnp.float32)

    def fill(i, _):
        onesbuf[pl.ds(i * L, L)] = one
        return 0

    lax.fori_loop(0, CE // L, fill, 0)
    plsc.subcore_barrier()

    def chunk_body(i, _):
        base = w * PERW + i * CE
        pltpu.sync_copy(dst_hbm.at[pl.ds(base, CE)], dstbuf)
        pltpu.sync_copy(onesbuf, acc_sh.at[dstbuf], add=True)
        return 0

    lax.fori_loop(0, NCHUNK, chunk_body, 0)
    plsc.subcore_barrier()
    pltpu.sync_copy(acc_sh.at[pl.ds(s * chunk, chunk)], stage)
    pltpu.sync_copy(stage, out_hbm.at[pl.ds(c * npad + s * chunk, chunk)])


def _deg(npad, dst):
    return pl.kernel(
        functools.partial(_deg_body, npad),
        out_type=jax.ShapeDtypeStruct((NC * npad,), jnp.float32),
        mesh=_mesh,
        compiler_params=_SC_PARAMS,
        scratch_types=[
            pltpu.VMEM_SHARED((npad,), jnp.float32),
            pltpu.VMEM((CE,), jnp.float32),
            pltpu.VMEM((CE,), jnp.int32),
            pltpu.VMEM((npad // NS,), jnp.float32),
        ],
    )(dst)


def _agg_body(npad, stride, g_hbm, src_hbm, dst_hbm, cnt_hbm, out_hbm, acc_sh,
              rows, srcbuf, dstbuf, cbuf, sem):
    c = lax.axis_index("c")
    s = lax.axis_index("s")
    w = s * NC + c
    rows_per = npad // NS
    z = jnp.zeros((L,), jnp.float32)

    def zrow(t, _):
        rows[t // 8, pl.ds((t % 8) * L, L)] = z
        return 0

    lax.fori_loop(0, CE * (H // L), zrow, 0)

    def zsh(i, _):
        pltpu.sync_copy(rows, acc_sh.at[pl.ds(s * rows_per + i * CE, CE)])
        return 0

    lax.fori_loop(0, rows_per // CE, zsh, 0)
    pltpu.sync_copy(cnt_hbm.at[pl.ds(w * L, L)], cbuf)
    plsc.subcore_barrier()
    csc = cbuf[pl.ds(0, L)][0]

    def chunk_body(i, _):
        @pl.when(i * CE < csc)
        def _():
            base = w * stride + i * CE
            pltpu.sync_copy(src_hbm.at[pl.ds(base, CE)], srcbuf)
            pltpu.sync_copy(dst_hbm.at[pl.ds(base, CE)], dstbuf)
            pltpu.async_copy(g_hbm.at[srcbuf], rows, sem).wait()
            pltpu.sync_copy(rows, acc_sh.at[dstbuf], add=True)

        return 0

    lax.fori_loop(0, NCHUNK, chunk_body, 0)
    plsc.subcore_barrier()

    def dump(i, _):
        pltpu.sync_copy(acc_sh.at[pl.ds(s * rows_per + i * CE, CE)], rows)
        pltpu.sync_copy(rows, out_hbm.at[c, pl.ds(s * rows_per + i * CE, CE)])
        return 0

    lax.fori_loop(0, rows_per // CE, dump, 0)


def _agg(npad, stride, g, src, dst, cnt):
    return pl.kernel(
        functools.partial(_agg_body, npad, stride),
        out_type=jax.ShapeDtypeStruct((NC, npad, H), jnp.float32),
        mesh=_mesh,
        compiler_params=_SC_PARAMS,
        scratch_types=[
            pltpu.VMEM_SHARED((npad, H), jnp.float32),
            pltpu.VMEM((CE, H), jnp.float32),
            pltpu.VMEM((CE,), jnp.int32),
            pltpu.VMEM((CE,), jnp.int32),
            pltpu.VMEM((L,), jnp.int32),
            pltpu.SemaphoreType.DMA,
        ],
    )(g, src, dst, cnt)


def _sagg_body(npad, stride, gp_hbm, src_hbm, dst_hbm, cnt_hbm, out_hbm,
               acc_sh, gp_l, valbuf, srcbuf, dstbuf, cbuf, stage):
    c = lax.axis_index("c")
    s = lax.axis_index("s")
    w = s * NC + c
    chunk = npad // NS
    pltpu.sync_copy(gp_hbm, gp_l)
    _zero_shared_1d(s, acc_sh, valbuf, npad)
    pltpu.sync_copy(cnt_hbm.at[pl.ds(w * L, L)], cbuf)
    plsc.subcore_barrier()
    csc = cbuf[pl.ds(0, L)][0]

    def chunk_body(i, _):
        @pl.when(i * CE < csc)
        def _():
            base = w * stride + i * CE
            pltpu.sync_copy(src_hbm.at[pl.ds(base, CE)], srcbuf)
            pltpu.sync_copy(dst_hbm.at[pl.ds(base, CE)], dstbuf)
            for g2 in range(CE // L):
                sidx = srcbuf[pl.ds(g2 * L, L)]
                valbuf[pl.ds(g2 * L, L)] = plsc.load_gather(gp_l, [sidx])
            pltpu.sync_copy(valbuf, acc_sh.at[dstbuf], add=True)

        return 0

    lax.fori_loop(0, NCHUNK, chunk_body, 0)
    plsc.subcore_barrier()
    pltpu.sync_copy(acc_sh.at[pl.ds(s * chunk, chunk)], stage)
    pltpu.sync_copy(stage, out_hbm.at[pl.ds(c * npad + s * chunk, chunk)])


def _sagg(npad, stride, gp, src, dst, cnt):
    return pl.kernel(
        functools.partial(_sagg_body, npad, stride),
        out_type=jax.ShapeDtypeStruct((NC * npad,), jnp.float32),
        mesh=_mesh,
        compiler_params=_SC_PARAMS,
        scratch_types=[
            pltpu.VMEM_SHARED((npad,), jnp.float32),
            pltpu.VMEM((npad,), jnp.float32),
            pltpu.VMEM((CE,), jnp.float32),
            pltpu.VMEM((CE,), jnp.int32),
            pltpu.VMEM((CE,), jnp.int32),
            pltpu.VMEM((L,), jnp.int32),
            pltpu.VMEM((npad // NS,), jnp.float32),
        ],
    )(gp, src, dst, cnt)


def _topk_body(npad, n, k, kpad, sacc_hbm, dinv_hbm, sb_hbm, perm_hbm,
               vals_hbm, map_hbm, sabuf, tmpbuf, dinvbuf, sbbuf, keybuf,
               scorebuf, mapbuf, posbuf, nodebuf, valbuf, zb_i, zb_f,
               cntstage, cntl, c1l, tiesl, cnt_sh, sem):
    c = lax.axis_index("c")
    s = lax.axis_index("s")
    chunk = npad // NS
    base = s * chunk
    kz = kpad // NS
    _zero1d(zb_i, kz)
    _zero1d(zb_f, kz)

    @pl.when(c == 0)
    def _():
        pltpu.sync_copy(zb_i, perm_hbm.at[pl.ds(s * kz, kz)])
        pltpu.sync_copy(zb_f, vals_hbm.at[pl.ds(s * kz, kz)])

    pltpu.sync_copy(sacc_hbm.at[pl.ds(base, chunk)], sabuf)
    pltpu.sync_copy(sacc_hbm.at[pl.ds(npad + base, chunk)], tmpbuf)
    pltpu.sync_copy(dinv_hbm.at[pl.ds(base, chunk)], dinvbuf)
    pltpu.sync_copy(sb_hbm.at[pl.ds(base, chunk)], sbbuf)
    iota = lax.iota(jnp.int32, L)
    topbit = jnp.full((L,), 0x80000000, jnp.uint32)

    def prep(t, _):
        sa = sabuf[pl.ds(t * L, L)] + tmpbuf[pl.ds(t * L, L)]
        sc = dinvbuf[pl.ds(t * L, L)] * sa + sbbuf[pl.ds(t * L, L)]
        gi = base + t * L + iota
        b = plsc.bitcast(sc, jnp.uint32)
        neg = b >= topbit
        key = jnp.where(neg, ~b, b | topbit)
        key = jnp.where(gi < n, key, jnp.zeros((L,), jnp.uint32))
        keybuf[pl.ds(t * L, L)] = key
        scorebuf[pl.ds(t * L, L)] = sc
        return 0

    lax.fori_loop(0, chunk // L, prep, 0)

    def count_pred(thr, mode):
        def cb(t, acc2):
            kv = keybuf[pl.ds(t * L, L)]
            if mode == 0:
                m = kv >= thr
            elif mode == 1:
                m = kv > thr
            else:
                m = kv == thr
            return acc2 + m.astype(jnp.int32)

        part = lax.fori_loop(0, chunk // L, cb, jnp.zeros((L,), jnp.int32))
        return plsc.cummax(lax.rev(plsc.cumsum(part), (0,)))

    def exch(v, dstbuf_l, p):
        del p
        cntstage[pl.ds(0, L)] = v
        plsc.subcore_barrier()
        pltpu.sync_copy(cntstage, cnt_sh.at[0, s])
        plsc.subcore_barrier()
        pltpu.sync_copy(cnt_sh.at[0], dstbuf_l)

    kk = jnp.full((L,), k, jnp.int32)

    def round_fn(r, lohi):
        lo, hi = lohi
        mid = lo + jnp.right_shift(hi - lo, jnp.full((L,), 1, jnp.uint32))
        cnt = count_pred(mid, 0)
        exch(cnt, cntl, jnp.bitwise_and(r, 1))
        tot = cntl[0, pl.ds(0, L)]
        for j in range(1, NS):
            tot = tot + cntl[j, pl.ds(0, L)]
        ge = tot >= kk
        return (jnp.where(ge, mid, lo), jnp.where(ge, hi, mid))

    lo0 = jnp.zeros((L,), jnp.uint32)
    hi0 = jnp.full((L,), 0xFFFFFFFF, jnp.uint32)
    tkey, _hi = lax.fori_loop(0, 32, round_fn, (lo0, hi0))

    exch(count_pred(tkey, 1), c1l, 0)
    exch(count_pred(tkey, 2), tiesl, 1)

    svec = jnp.zeros((L,), jnp.int32) + s
    zi = jnp.zeros((L,), jnp.int32)
    c1tot = zi
    for j in range(NS):
        c1tot = c1tot + c1l[j, pl.ds(0, L)]
    trun = zi
    offrun = zi
    mytake = zi
    for j in range(NS):
        c1j = c1l[j, pl.ds(0, L)]
        tj = tiesl[j, pl.ds(0, L)]
        takej = jnp.clip(kk - c1tot - trun, 0, tj)
        jv = jnp.full((L,), j, jnp.int32)
        offrun = offrun + jnp.where(jv < svec, c1j + takej, zi)
        mytake = mytake + jnp.where(jv == svec, takej, zi)
        trun = trun + tj
    off_w = offrun
    take_w = mytake

    ktrash = jnp.full((L,), k, jnp.int32) + s
    negone = jnp.full((L,), -1, jnp.int32)

    def comp(t, cur):
        cursor, tiecur = cur
        kv = keybuf[pl.ds(t * L, L)]
        sc = scorebuf[pl.ds(t * L, L)]
        gi = base + t * L + iota
        gt = kv > tkey
        eq = kv == tkey
        eqi = eq.astype(jnp.int32)
        eqexcl = plsc.cumsum(eqi) - eqi
        tierank = tiecur + eqexcl
        sel = gt | (eq & (tierank < take_w))
        seli = sel.astype(jnp.int32)
        selexcl = plsc.cumsum(seli) - seli
        rank = off_w + cursor + selexcl
        mapbuf[pl.ds(t * L, L)] = jnp.where(sel, rank, negone)
        posbuf[t // 2, pl.ds((t % 2) * L, L)] = jnp.where(sel, rank, ktrash)
        nodebuf[t // 2, pl.ds((t % 2) * L, L)] = gi
        valbuf[t // 2, pl.ds((t % 2) * L, L)] = sc
        return (cursor + plsc.all_reduce_population_count(sel),
                tiecur + plsc.all_reduce_population_count(eq))

    lax.fori_loop(0, chunk // L, comp, (zi, zi))
    plsc.subcore_barrier()

    @pl.when(c == 0)
    def _():
        pltpu.sync_copy(mapbuf, map_hbm.at[pl.ds(base, chunk)])
        for j in range(chunk // SW):
            pltpu.async_copy(nodebuf.at[j], perm_hbm.at[posbuf.at[j]],
                             sem).wait()
            pltpu.async_copy(valbuf.at[j], vals_hbm.at[posbuf.at[j]],
                             sem).wait()


def _topk(npad, n, k, kpad, sacc, dinv1d, sb1d):
    chunk = npad // NS
    return pl.kernel(
        functools.partial(_topk_body, npad, n, k, kpad),
        out_type=(
            jax.ShapeDtypeStruct((kpad,), jnp.int32),
            jax.ShapeDtypeStruct((kpad,), jnp.float32),
            jax.ShapeDtypeStruct((npad,), jnp.int32),
        ),
        mesh=_mesh,
        compiler_params=_SC_PARAMS,
        scratch_types=[
            pltpu.VMEM((chunk,), jnp.float32),
            pltpu.VMEM((chunk,), jnp.float32),
            pltpu.VMEM((chunk,), jnp.float32),
            pltpu.VMEM((chunk,), jnp.float32),
            pltpu.VMEM((chunk,), jnp.uint32),
            pltpu.VMEM((chunk,), jnp.float32),
            pltpu.VMEM((chunk,), jnp.int32),
            pltpu.VMEM((chunk // SW, SW), jnp.int32),
            pltpu.VMEM((chunk // SW, SW), jnp.int32),
            pltpu.VMEM((chunk // SW, SW), jnp.float32),
            pltpu.VMEM((kpad // NS,), jnp.int32),
            pltpu.VMEM((kpad // NS,), jnp.float32),
            pltpu.VMEM((L,), jnp.int32),
            pltpu.VMEM((NS, L), jnp.int32),
            pltpu.VMEM((NS, L), jnp.int32),
            pltpu.VMEM((NS, L), jnp.int32),
            pltpu.VMEM_SHARED((2, NS, L), jnp.int32),
            pltpu.SemaphoreType.DMA,
        ],
    )(sacc, dinv1d, sb1d)


def _gather_body(kpad, h_hbm, perm_hbm, out_hbm, idxbuf, rows, sem):
    c = lax.axis_index("c")
    s = lax.axis_index("s")
    w = s * NC + c
    kw = kpad // NW

    def body(i, _):
        base = w * kw + i * CEG
        pltpu.sync_copy(perm_hbm.at[pl.ds(base, CEG)], idxbuf)
        pltpu.async_copy(h_hbm.at[idxbuf], rows, sem).wait()
        pltpu.sync_copy(rows, out_hbm.at[pl.ds(base, CEG)])
        return 0

    lax.fori_loop(0, kw // CEG, body, 0)


def _gatherrows(kpad, h, perm):
    return pl.kernel(
        functools.partial(_gather_body, kpad),
        out_type=jax.ShapeDtypeStruct((kpad, H), jnp.float32),
        mesh=_mesh,
        compiler_params=_SC_PARAMS,
        scratch_types=[
            pltpu.VMEM((CEG,), jnp.int32),
            pltpu.VMEM((CEG, H), jnp.float32),
            pltpu.SemaphoreType.DMA,
        ],
    )(h, perm)


def _remap_body(npad, in_stride, k, kpad, map_hbm, src_hbm, dst_hbm, incnt_hbm,
                order_hbm, src2_hbm, dst2_hbm, deg_hbm, cnt_hbm, acc_sh, s2_sh,
                d2_sh, map_l, onesbuf, srcbuf, dstbuf, s2buf, d2buf, posbuf,
                fsrcbuf, fdstbuf, cstage, icbuf, dstage, stage):
    del order_hbm
    c = lax.axis_index("c")
    s = lax.axis_index("s")
    w = s * NC + c
    chunk = kpad // NS
    lbase = s * SEG
    pltpu.sync_copy(map_hbm, map_l)
    _zero_shared_1d(s, acc_sh, onesbuf, kpad)
    one = jnp.ones((L,), jnp.float32)
    ktr = jnp.full((L,), k, jnp.int32) + w
    zi = jnp.zeros((L,), jnp.int32)

    def fill(i, _):
        onesbuf[pl.ds(i * L, L)] = one
        fsrcbuf[pl.ds(i * L, L)] = zi
        fdstbuf[pl.ds(i * L, L)] = ktr
        return 0

    lax.fori_loop(0, CE // L, fill, 0)
    pltpu.sync_copy(incnt_hbm.at[pl.ds(w * L, L)], icbuf)
    plsc.subcore_barrier()
    icsc = icbuf[pl.ds(0, L)][0]
    lbv = zi + lbase
    dumpv = zi + (lbase + SEG - 1)

    def chunk_body(i, cnt):
        @pl.when(i * CE < icsc)
        def _():
            base = w * in_stride + i * CE
            pltpu.sync_copy(src_hbm.at[pl.ds(base, CE)], srcbuf)
            pltpu.sync_copy(dst_hbm.at[pl.ds(base, CE)], dstbuf)
            ccnt = zi
            for g2 in range(CE // L):
                sidx = srcbuf[pl.ds(g2 * L, L)]
                didx = dstbuf[pl.ds(g2 * L, L)]
                ms = plsc.load_gather(map_l, [sidx])
                md = plsc.load_gather(map_l, [didx])
                valid = (ms >= 0) & (md >= 0)
                vi = valid.astype(jnp.int32)
                excl = plsc.cumsum(vi) - vi
                pos = lbv + cnt + ccnt + excl
                posbuf[pl.ds(g2 * L, L)] = jnp.where(valid, pos, dumpv)
                s2buf[pl.ds(g2 * L, L)] = jnp.maximum(ms, zi)
                d2buf[pl.ds(g2 * L, L)] = jnp.where(valid, md, ktr)
                ccnt = ccnt + plsc.all_reduce_population_count(valid)
            cstage[pl.ds(0, L)] = ccnt
            pltpu.sync_copy(s2buf, s2_sh.at[posbuf])
            pltpu.sync_copy(d2buf, d2_sh.at[posbuf])
            pltpu.sync_copy(onesbuf, acc_sh.at[d2buf], add=True)

        cc = cstage[pl.ds(0, L)]
        pv = (zi + icsc) > (zi + i * CE)
        return cnt + jnp.where(pv, cc, zi)

    cstage[pl.ds(0, L)] = zi
    cnt = lax.fori_loop(0, NCHUNK, chunk_body, zi)
    cstage[pl.ds(0, L)] = cnt
    iota = lax.iota(jnp.int32, L)
    for i in range(CE // L):
        posbuf[pl.ds(i * L, L)] = lbv + cnt + iota + (i * L)
    pltpu.sync_copy(fsrcbuf, s2_sh.at[posbuf])
    pltpu.sync_copy(fdstbuf, d2_sh.at[posbuf])
    pltpu.sync_copy(cstage, cnt_hbm.at[pl.ds(w * L, L)])
    csc = cstage[pl.ds(0, L)][0]

    def dchunk(i, _):
        @pl.when(i * DC < csc + CE)
        def _():
            pltpu.sync_copy(s2_sh.at[pl.ds(lbase + i * DC, DC)], dstage)
            pltpu.sync_copy(dstage, src2_hbm.at[pl.ds(w * SEG + i * DC, DC)])
            pltpu.sync_copy(d2_sh.at[pl.ds(lbase + i * DC, DC)], dstage)
            pltpu.sync_copy(dstage, dst2_hbm.at[pl.ds(w * SEG + i * DC, DC)])

        return 0

    lax.fori_loop(0, SEG // DC, dchunk, 0)
    plsc.subcore_barrier()
    pltpu.sync_copy(acc_sh.at[pl.ds(s * chunk, chunk)], stage)
    pltpu.sync_copy(stage, deg_hbm.at[pl.ds(c * kpad + s * chunk, chunk)])


def _remap(npad, in_stride, k, kpad, mapping, src, dst, incnt, order):
    return pl.kernel(
        functools.partial(_remap_body, npad, in_stride, k, kpad),
        out_type=(
            jax.ShapeDtypeStruct((NSEG,), jnp.int32),
            jax.ShapeDtypeStruct((NSEG,), jnp.int32),
            jax.ShapeDtypeStruct((NC * kpad,), jnp.float32),
            jax.ShapeDtypeStruct((NW * L,), jnp.int32),
        ),
        mesh=_mesh,
        compiler_params=_SC_PARAMS,
        scratch_types=[
            pltpu.VMEM_SHARED((kpad,), jnp.float32),
            pltpu.VMEM_SHARED((NS * SEG,), jnp.int32),
            pltpu.VMEM_SHARED((NS * SEG,), jnp.int32),
            pltpu.VMEM((npad,), jnp.int32),
            pltpu.VMEM((CE,), jnp.float32),
            pltpu.VMEM((CE,), jnp.int32),
            pltpu.VMEM((CE,), jnp.int32),
            pltpu.VMEM((CE,), jnp.int32),
            pltpu.VMEM((CE,), jnp.int32),
            pltpu.VMEM((CE,), jnp.int32),
            pltpu.VMEM((CE,), jnp.int32),
            pltpu.VMEM((CE,), jnp.int32),
            pltpu.VMEM((L,), jnp.int32),
            pltpu.VMEM((L,), jnp.int32),
            pltpu.VMEM((DC,), jnp.int32),
            pltpu.VMEM((kpad // NS,), jnp.float32),
        ],
    )(mapping, src, dst, incnt, order)


def _tc1_body(x_ref, w_ref, deg_ref, h1_ref, g_ref, dinv_ref):
    deg = deg_ref[...]
    dinv = lax.rsqrt(deg[0] + deg[1] + 1.0)
    h1 = jnp.dot(x_ref[...], w_ref[...], preferred_element_type=jnp.float32)
    h1_ref[...] = h1
    g_ref[...] = dinv * h1
    dinv_ref[...] = dinv


def _tc1(x_p, w, degacc3):
    npad = x_p.shape[0]
    return pl.pallas_call(
        _tc1_body,
        out_shape=(
            jax.ShapeDtypeStruct((npad, H), jnp.float32),
            jax.ShapeDtypeStruct((npad, H), jnp.float32),
            jax.ShapeDtypeStruct((npad, 1), jnp.float32),
        ),
    )(x_p, w, degacc3)


def _tc2_body(S_ref, h1_ref, dinv_ref, b_ref, wp_ref, bp_ref, h_ref, sb_ref,
              gp_ref):
    S = S_ref[...]
    dinv = dinv_ref[...]
    h1 = h1_ref[...]
    A = dinv * (S[0] + S[1]) + (dinv * dinv) * h1 + b_ref[...]
    h = jnp.maximum(A, 0.0)
    h_ref[...] = h
    hp = jnp.dot(h, wp_ref[...], preferred_element_type=jnp.float32)
    sb_ref[...] = (dinv * dinv) * hp + bp_ref[...]
    gp_ref[...] = dinv * hp


def _tc2(S, h1, dinv, b2d, wp, bp2d):
    npad = h1.shape[0]
    return pl.pallas_call(
        _tc2_body,
        out_shape=(
            jax.ShapeDtypeStruct((npad, H), jnp.float32),
            jax.ShapeDtypeStruct((npad, 1), jnp.float32),
            jax.ShapeDtypeStruct((npad, 1), jnp.float32),
        ),
    )(S, h1, dinv, b2d, wp, bp2d)


def _readout_piece(kq, kpad, xn, vals):
    gate = jnp.tanh(vals)
    rowid = lax.broadcasted_iota(jnp.int32, (kpad, 1), 0)
    mask = rowid < kq
    xg = jnp.where(mask, xn * gate, 0.0)
    rmax = jnp.max(jnp.where(mask, xn * gate, -jnp.inf), axis=0,
                   keepdims=True)
    rmean = jnp.sum(xg, axis=0, keepdims=True) / kq
    return xg, jnp.concatenate([rmax, rmean], axis=1)


def _tc3_body(kq, xn_ref, vals_ref, wn_ref, deg_ref, r_ref, h1_ref, g_ref,
              dinv_ref):
    kpad = xn_ref.shape[0]
    xg, r = _readout_piece(kq, kpad, xn_ref[...], vals_ref[...])
    r_ref[...] = r
    deg = deg_ref[...]
    dinv = lax.rsqrt(deg[0] + deg[1] + 1.0)
    h1 = jnp.dot(xg, wn_ref[...], preferred_element_type=jnp.float32)
    h1_ref[...] = h1
    g_ref[...] = dinv * h1
    dinv_ref[...] = dinv


def _tc3(kq, xn, vals2d, wn, degacc3):
    kpad = xn.shape[0]
    return pl.pallas_call(
        functools.partial(_tc3_body, kq),
        out_shape=(
            jax.ShapeDtypeStruct((1, 2 * H), jnp.float32),
            jax.ShapeDtypeStruct((kpad, H), jnp.float32),
            jax.ShapeDtypeStruct((kpad, H), jnp.float32),
            jax.ShapeDtypeStruct((kpad, 1), jnp.float32),
        ),
    )(xn, vals2d, wn, degacc3)


def _readout_body(kq, xn_ref, vals_ref, r_ref):
    kpad = xn_ref.shape[0]
    _, r = _readout_piece(kq, kpad, xn_ref[...], vals_ref[...])
    r_ref[...] = r


def _readout(kq, xn, vals2d):
    return pl.pallas_call(
        functools.partial(_readout_body, kq),
        out_shape=jax.ShapeDtypeStruct((1, 2 * H), jnp.float32),
    )(xn, vals2d)


def _head_body(r1_ref, r2_ref, r3_ref, w1_ref, b1_ref, w2_ref, b2_ref, w3_ref,
               b3_ref, o_ref):
    z = r1_ref[...] + r2_ref[...] + r3_ref[...]
    z = jnp.maximum(
        jnp.dot(z, w1_ref[...], preferred_element_type=jnp.float32)
        + b1_ref[...], 0.0)
    z = jnp.maximum(
        jnp.dot(z, w2_ref[...], preferred_element_type=jnp.float32)
        + b2_ref[...], 0.0)
    z = jnp.dot(z, w3_ref[...], preferred_element_type=jnp.float32) + b3_ref[...]
    m = jnp.max(z, axis=1, keepdims=True)
    o_ref[...] = z - m - jnp.log(
        jnp.sum(jnp.exp(z - m), axis=1, keepdims=True))


def _head(r1, r2, r3, wl1, bl1, wl2, bl2, wl3, bl3):
    return pl.pallas_call(
        _head_body,
        out_shape=jax.ShapeDtypeStruct((1, bl3.shape[1]), jnp.float32),
    )(r1, r2, r3, wl1, bl1, wl2, bl2, wl3, bl3)


def kernel(x, edge_index, batch, W1, b1, Wp1, bp1, W2, b2, Wp2, bp2, W3, b3,
           Wp3, bp3, Wl1, bl1, Wl2, bl2, Wl3, bl3):
    src = edge_index[0].astype(jnp.int32)
    dst = edge_index[1].astype(jnp.int32)
    x_p = jnp.pad(x, ((0, NP1 - N0), (0, 0)))
    cnt_full = jnp.full((NW * L,), PERW, jnp.int32)

    degacc = _deg(NP1, dst)
    h11, g1, dinv1 = _tc1(x_p, W1, degacc.reshape(NC, NP1, 1))
    S1 = _agg(NP1, PERW, g1, src, dst, cnt_full)
    h1o, sb1, gp1 = _tc2(S1, h11, dinv1, b1.reshape(1, H), Wp1,
                         bp1.reshape(1, 1))
    sacc1 = _sagg(NP1, PERW, gp1.reshape(NP1), src, dst, cnt_full)
    perm1, vals1, map1 = _topk(NP1, N0, N0 // 2, NP2, sacc1,
                               dinv1.reshape(NP1), sb1.reshape(NP1))
    xn1 = _gatherrows(NP2, h1o, perm1)
    src2, dst2, deg2, cnt2 = _remap(NP1, PERW, N0 // 2, NP2, map1, src, dst,
                                    cnt_full, xn1)
    r1, h12, g2, dinv2 = _tc3(N0 // 2, xn1, vals1.reshape(NP2, 1), W2,
                              deg2.reshape(NC, NP2, 1))

    S2 = _agg(NP2, SEG, g2, src2, dst2, cnt2)
    h2o, sb2, gp2 = _tc2(S2, h12, dinv2, b2.reshape(1, H), Wp2,
                         bp2.reshape(1, 1))
    sacc2 = _sagg(NP2, SEG, gp2.reshape(NP2), src2, dst2, cnt2)
    perm2, vals2, map2 = _topk(NP2, N0 // 2, N0 // 4, NP3, sacc2,
                               dinv2.reshape(NP2), sb2.reshape(NP2))
    xn2 = _gatherrows(NP3, h2o, perm2)
    src3, dst3, deg3, cnt3 = _remap(NP2, SEG, N0 // 4, NP3, map2, src2, dst2,
                                    cnt2, xn2)
    r2, h13, g3, dinv3 = _tc3(N0 // 4, xn2, vals2.reshape(NP3, 1), W3,
                              deg3.reshape(NC, NP3, 1))

    S3 = _agg(NP3, SEG, g3, src3, dst3, cnt3)
    h3o, sb3, gp3 = _tc2(S3, h13, dinv3, b3.reshape(1, H), Wp3,
                         bp3.reshape(1, 1))
    sacc3 = _sagg(NP3, SEG, gp3.reshape(NP3), src3, dst3, cnt3)
    perm3, vals3, _map3 = _topk(NP3, N0 // 4, N0 // 8, NP4, sacc3,
                                dinv3.reshape(NP3), sb3.reshape(NP3))
    xn3 = _gatherrows(NP4, h3o, perm3)
    r3 = _readout(N0 // 8, xn3, vals3.reshape(NP4, 1))

    return _head(r1, r2, r3, Wl1, bl1.reshape(1, H), Wl2,
                 bl2.reshape(1, H // 2), Wl3, bl3.reshape(1, 10))


# speedup vs baseline: 1.2527x; 1.2527x over previous
import functools

import jax
import jax.numpy as jnp
from jax import lax
from jax.experimental import pallas as pl
from jax.experimental.pallas import tpu as pltpu
from jax.experimental.pallas import tpu_sc as plsc

NC = 2
NS = 16
L = 16
NW = NC * NS
E = 320000
H = 128
N0 = 10000
CE = 80
PERW = E // NW
NCHUNK = PERW // CE
SW = 32
CEG = 40

DC = 640
SEG = 10240
NSEG = NW * SEG

NP1, NP2, NP3, NP4 = 10240, 5120, 2560, 1280

_mesh = plsc.VectorSubcoreMesh(core_axis_name="c", subcore_axis_name="s")
_SC_PARAMS = pltpu.CompilerParams(needs_layout_passes=False)


def _zero1d(ref, m):
    z = jnp.zeros((L,), ref.dtype)

    def body(i, _):
        ref[pl.ds(i * L, L)] = z
        return 0

    lax.fori_loop(0, m // L, body, 0)


def _zero_shared_1d(s, acc_sh, zbuf, m):
    _zero1d(zbuf, CE)
    chunk = m // NS

    def body(i, _):
        pltpu.sync_copy(zbuf, acc_sh.at[pl.ds(s * chunk + i * CE, CE)])
        return 0

    lax.fori_loop(0, chunk // CE, body, 0)


def _deg_body(npad, dst_hbm, out_hbm, acc_sh, onesbuf, dstbuf, stage):
    c = lax.axis_index("c")
    s = lax.axis_index("s")
    w = s * NC + c
    chunk = npad // NS
    _zero_shared_1d(s, acc_sh, onesbuf, npad)
    one = jnp.ones((L,), jnp.float32)

    def fill(i, _):
        onesbuf[pl.ds(i * L, L)] = one
        return 0

    lax.fori_loop(0, CE // L, fill, 0)
    plsc.subcore_barrier()

    def chunk_body(i, _):
        base = w * PERW + i * CE
        pltpu.sync_copy(dst_hbm.at[pl.ds(base, CE)], dstbuf)
        pltpu.sync_copy(onesbuf, acc_sh.at[dstbuf], add=True)
        return 0

    lax.fori_loop(0, NCHUNK, chunk_body, 0)
    plsc.subcore_barrier()
    pltpu.sync_copy(acc_sh.at[pl.ds(s * chunk, chunk)], stage)
    pltpu.sync_copy(stage, out_hbm.at[pl.ds(c * npad + s * chunk, chunk)])


def _deg(npad, dst):
    return pl.kernel(
        functools.partial(_deg_body, npad),
        out_type=jax.ShapeDtypeStruct((NC * npad,), jnp.float32),
        mesh=_mesh,
        compiler_params=_SC_PARAMS,
        scratch_types=[
            pltpu.VMEM_SHARED((npad,), jnp.float32),
            pltpu.VMEM((CE,), jnp.float32),
            pltpu.VMEM((CE,), jnp.int32),
            pltpu.VMEM((npad // NS,), jnp.float32),
        ],
    )(dst)


def _agg_body(npad, stride, g_hbm, src_hbm, dst_hbm, cnt_hbm, out_hbm, acc_sh,
              rows, srcbuf, dstbuf, cbuf, sem):
    c = lax.axis_index("c")
    s = lax.axis_index("s")
    w = s * NC + c
    rows_per = npad // NS
    z = jnp.zeros((L,), jnp.float32)

    def zrow(t, _):
        rows[t // 8, pl.ds((t % 8) * L, L)] = z
        return 0

    lax.fori_loop(0, CE * (H // L), zrow, 0)

    def zsh(i, _):
        pltpu.sync_copy(rows, acc_sh.at[pl.ds(s * rows_per + i * CE, CE)])
        return 0

    lax.fori_loop(0, rows_per // CE, zsh, 0)
    pltpu.sync_copy(cnt_hbm.at[pl.ds(w * L, L)], cbuf)
    plsc.subcore_barrier()
    csc = cbuf[pl.ds(0, L)][0]

    def chunk_body(i, _):
        @pl.when(i * CE < csc)
        def _():
            base = w * stride + i * CE
            pltpu.sync_copy(src_hbm.at[pl.ds(base, CE)], srcbuf)
            pltpu.sync_copy(dst_hbm.at[pl.ds(base, CE)], dstbuf)
            pltpu.async_copy(g_hbm.at[srcbuf], rows, sem).wait()
            pltpu.sync_copy(rows, acc_sh.at[dstbuf], add=True)

        return 0

    lax.fori_loop(0, NCHUNK, chunk_body, 0)
    plsc.subcore_barrier()

    def dump(i, _):
        pltpu.sync_copy(acc_sh.at[pl.ds(s * rows_per + i * CE, CE)], rows)
        pltpu.sync_copy(rows, out_hbm.at[c, pl.ds(s * rows_per + i * CE, CE)])
        return 0

    lax.fori_loop(0, rows_per // CE, dump, 0)


def _agg(npad, stride, g, src, dst, cnt):
    return pl.kernel(
        functools.partial(_agg_body, npad, stride),
        out_type=jax.ShapeDtypeStruct((NC, npad, H), jnp.float32),
        mesh=_mesh,
        compiler_params=_SC_PARAMS,
        scratch_types=[
            pltpu.VMEM_SHARED((npad, H), jnp.float32),
            pltpu.VMEM((CE, H), jnp.float32),
            pltpu.VMEM((CE,), jnp.int32),
            pltpu.VMEM((CE,), jnp.int32),
            pltpu.VMEM((L,), jnp.int32),
            pltpu.SemaphoreType.DMA,
        ],
    )(g, src, dst, cnt)


def _sagg_body(npad, stride, gp_hbm, src_hbm, dst_hbm, cnt_hbm, out_hbm,
               acc_sh, gp_l, valbuf, srcbuf, dstbuf, cbuf, stage):
    c = lax.axis_index("c")
    s = lax.axis_index("s")
    w = s * NC + c
    chunk = npad // NS
    pltpu.sync_copy(gp_hbm, gp_l)
    _zero_shared_1d(s, acc_sh, valbuf, npad)
    pltpu.sync_copy(cnt_hbm.at[pl.ds(w * L, L)], cbuf)
    plsc.subcore_barrier()
    csc = cbuf[pl.ds(0, L)][0]

    def chunk_body(i, _):
        @pl.when(i * CE < csc)
        def _():
            base = w * stride + i * CE
            pltpu.sync_copy(src_hbm.at[pl.ds(base, CE)], srcbuf)
            pltpu.sync_copy(dst_hbm.at[pl.ds(base, CE)], dstbuf)
            for g2 in range(CE // L):
                sidx = srcbuf[pl.ds(g2 * L, L)]
                valbuf[pl.ds(g2 * L, L)] = plsc.load_gather(gp_l, [sidx])
            pltpu.sync_copy(valbuf, acc_sh.at[dstbuf], add=True)

        return 0

    lax.fori_loop(0, NCHUNK, chunk_body, 0)
    plsc.subcore_barrier()
    pltpu.sync_copy(acc_sh.at[pl.ds(s * chunk, chunk)], stage)
    pltpu.sync_copy(stage, out_hbm.at[pl.ds(c * npad + s * chunk, chunk)])


def _sagg(npad, stride, gp, src, dst, cnt):
    return pl.kernel(
        functools.partial(_sagg_body, npad, stride),
        out_type=jax.ShapeDtypeStruct((NC * npad,), jnp.float32),
        mesh=_mesh,
        compiler_params=_SC_PARAMS,
        scratch_types=[
            pltpu.VMEM_SHARED((npad,), jnp.float32),
            pltpu.VMEM((npad,), jnp.float32),
            pltpu.VMEM((CE,), jnp.float32),
            pltpu.VMEM((CE,), jnp.int32),
            pltpu.VMEM((CE,), jnp.int32),
            pltpu.VMEM((L,), jnp.int32),
            pltpu.VMEM((npad // NS,), jnp.float32),
        ],
    )(gp, src, dst, cnt)


def _topk_body(npad, n, k, kpad, sacc_hbm, dinv_hbm, sb_hbm, perm_hbm,
               vals_hbm, map_hbm, sabuf, tmpbuf, dinvbuf, sbbuf, keybuf,
               scorebuf, mapbuf, posbuf, nodebuf, valbuf, zb_i, zb_f,
               cntstage, cntl, c1l, tiesl, cnt_sh, sem):
    c = lax.axis_index("c")
    s = lax.axis_index("s")
    chunk = npad // NS
    base = s * chunk
    kz = kpad // NS
    _zero1d(zb_i, kz)
    _zero1d(zb_f, kz)

    @pl.when(c == 0)
    def _():
        pltpu.sync_copy(zb_i, perm_hbm.at[pl.ds(s * kz, kz)])
        pltpu.sync_copy(zb_f, vals_hbm.at[pl.ds(s * kz, kz)])

    pltpu.sync_copy(sacc_hbm.at[pl.ds(base, chunk)], sabuf)
    pltpu.sync_copy(sacc_hbm.at[pl.ds(npad + base, chunk)], tmpbuf)
    pltpu.sync_copy(dinv_hbm.at[pl.ds(base, chunk)], dinvbuf)
    pltpu.sync_copy(sb_hbm.at[pl.ds(base, chunk)], sbbuf)
    iota = lax.iota(jnp.int32, L)
    topbit = jnp.full((L,), 0x80000000, jnp.uint32)

    def prep(t, _):
        sa = sabuf[pl.ds(t * L, L)] + tmpbuf[pl.ds(t * L, L)]
        sc = dinvbuf[pl.ds(t * L, L)] * sa + sbbuf[pl.ds(t * L, L)]
        gi = base + t * L + iota
        b = plsc.bitcast(sc, jnp.uint32)
        neg = b >= topbit
        key = jnp.where(neg, ~b, b | topbit)
        key = jnp.where(gi < n, key, jnp.zeros((L,), jnp.uint32))
        keybuf[pl.ds(t * L, L)] = key
        scorebuf[pl.ds(t * L, L)] = sc
        return 0

    lax.fori_loop(0, chunk // L, prep, 0)

    def count_pred(thr, mode):
        part = jnp.zeros((L,), jnp.int32)
        for t in range(chunk // L):
            kv = keybuf[pl.ds(t * L, L)]
            if mode == 0:
                m = kv >= thr
            elif mode == 1:
                m = kv > thr
            else:
                m = kv == thr
            part = part + m.astype(jnp.int32)
        tot = jnp.zeros((L,), jnp.int32)
        one = jnp.full((L,), 1, jnp.int32)
        for b in range(6):
            bit = (jnp.right_shift(part, jnp.full((L,), b, jnp.int32))
                   & one) > 0
            tot = tot + jnp.left_shift(
                plsc.all_reduce_population_count(bit),
                jnp.full((L,), b, jnp.int32))
        return tot

    def exch(v, dstbuf_l):
        cntstage[pl.ds(0, L)] = v
        plsc.subcore_barrier()
        pltpu.sync_copy(cntstage, cnt_sh.at[0, s])
        plsc.subcore_barrier()
        pltpu.sync_copy(cnt_sh.at[0], dstbuf_l)

    kk = jnp.full((L,), k, jnp.int32)

    def round_fn(r, lohi):
        lo, hi = lohi
        mid = lo + jnp.right_shift(hi - lo, jnp.full((L,), 1, jnp.uint32))
        cnt = count_pred(mid, 0)
        exch(cnt, cntl)
        tot = cntl[0, pl.ds(0, L)]
        for j in range(1, NS):
            tot = tot + cntl[j, pl.ds(0, L)]
        ge = tot >= kk
        return (jnp.where(ge, mid, lo), jnp.where(ge, hi, mid))

    lo0 = jnp.zeros((L,), jnp.uint32)
    hi0 = jnp.full((L,), 0xFFFFFFFF, jnp.uint32)
    tkey, _hi = lax.fori_loop(0, 32, round_fn, (lo0, hi0))

    exch(count_pred(tkey, 1), c1l)
    exch(count_pred(tkey, 2), tiesl)

    svec = jnp.zeros((L,), jnp.int32) + s
    zi = jnp.zeros((L,), jnp.int32)
    c1tot = zi
    for j in range(NS):
        c1tot = c1tot + c1l[j, pl.ds(0, L)]
    trun = zi
    offrun = zi
    mytake = zi
    for j in range(NS):
        c1j = c1l[j, pl.ds(0, L)]
        tj = tiesl[j, pl.ds(0, L)]
        takej = jnp.clip(kk - c1tot - trun, 0, tj)
        jv = jnp.full((L,), j, jnp.int32)
        offrun = offrun + jnp.where(jv < svec, c1j + takej, zi)
        mytake = mytake + jnp.where(jv == svec, takej, zi)
        trun = trun + tj
    off_w = offrun
    take_w = mytake

    ktrash = jnp.full((L,), k, jnp.int32) + s
    negone = jnp.full((L,), -1, jnp.int32)

    def comp(t, cur):
        cursor, tiecur = cur
        kv = keybuf[pl.ds(t * L, L)]
        sc = scorebuf[pl.ds(t * L, L)]
        gi = base + t * L + iota
        gt = kv > tkey
        eq = kv == tkey
        eqi = eq.astype(jnp.int32)
        eqexcl = plsc.cumsum(eqi) - eqi
        tierank = tiecur + eqexcl
        sel = gt | (eq & (tierank < take_w))
        seli = sel.astype(jnp.int32)
        selexcl = plsc.cumsum(seli) - seli
        rank = off_w + cursor + selexcl
        mapbuf[pl.ds(t * L, L)] = jnp.where(sel, rank, negone)
        posbuf[t // 2, pl.ds((t % 2) * L, L)] = jnp.where(sel, rank, ktrash)
        nodebuf[t // 2, pl.ds((t % 2) * L, L)] = gi
        valbuf[t // 2, pl.ds((t % 2) * L, L)] = sc
        return (cursor + plsc.all_reduce_population_count(sel),
                tiecur + plsc.all_reduce_population_count(eq))

    lax.fori_loop(0, chunk // L, comp, (zi, zi))
    plsc.subcore_barrier()

    @pl.when(c == 0)
    def _():
        pltpu.sync_copy(mapbuf, map_hbm.at[pl.ds(base, chunk)])
        for j in range(chunk // SW):
            pltpu.async_copy(nodebuf.at[j], perm_hbm.at[posbuf.at[j]],
                             sem).wait()
            pltpu.async_copy(valbuf.at[j], vals_hbm.at[posbuf.at[j]],
                             sem).wait()


def _topk(npad, n, k, kpad, sacc, dinv1d, sb1d):
    chunk = npad // NS
    return pl.kernel(
        functools.partial(_topk_body, npad, n, k, kpad),
        out_type=(
            jax.ShapeDtypeStruct((kpad,), jnp.int32),
            jax.ShapeDtypeStruct((kpad,), jnp.float32),
            jax.ShapeDtypeStruct((npad,), jnp.int32),
        ),
        mesh=_mesh,
        compiler_params=_SC_PARAMS,
        scratch_types=[
            pltpu.VMEM((chunk,), jnp.float32),
            pltpu.VMEM((chunk,), jnp.float32),
            pltpu.VMEM((chunk,), jnp.float32),
            pltpu.VMEM((chunk,), jnp.float32),
            pltpu.VMEM((chunk,), jnp.uint32),
            pltpu.VMEM((chunk,), jnp.float32),
            pltpu.VMEM((chunk,), jnp.int32),
            pltpu.VMEM((chunk // SW, SW), jnp.int32),
            pltpu.VMEM((chunk // SW, SW), jnp.int32),
            pltpu.VMEM((chunk // SW, SW), jnp.float32),
            pltpu.VMEM((kpad // NS,), jnp.int32),
            pltpu.VMEM((kpad // NS,), jnp.float32),
            pltpu.VMEM((L,), jnp.int32),
            pltpu.VMEM((NS, L), jnp.int32),
            pltpu.VMEM((NS, L), jnp.int32),
            pltpu.VMEM((NS, L), jnp.int32),
            pltpu.VMEM_SHARED((2, NS, L), jnp.int32),
            pltpu.SemaphoreType.DMA,
        ],
    )(sacc, dinv1d, sb1d)


def _gather_body(kpad, h_hbm, perm_hbm, out_hbm, idxbuf, rows, sem):
    c = lax.axis_index("c")
    s = lax.axis_index("s")
    w = s * NC + c
    kw = kpad // NW

    def body(i, _):
        base = w * kw + i * CEG
        pltpu.sync_copy(perm_hbm.at[pl.ds(base, CEG)], idxbuf)
        pltpu.async_copy(h_hbm.at[idxbuf], rows, sem).wait()
        pltpu.sync_copy(rows, out_hbm.at[pl.ds(base, CEG)])
        return 0

    lax.fori_loop(0, kw // CEG, body, 0)


def _gatherrows(kpad, h, perm):
    return pl.kernel(
        functools.partial(_gather_body, kpad),
        out_type=jax.ShapeDtypeStruct((kpad, H), jnp.float32),
        mesh=_mesh,
        compiler_params=_SC_PARAMS,
        scratch_types=[
            pltpu.VMEM((CEG,), jnp.int32),
            pltpu.VMEM((CEG, H), jnp.float32),
            pltpu.SemaphoreType.DMA,
        ],
    )(h, perm)


def _remap_body(npad, in_stride, k, kpad, map_hbm, src_hbm, dst_hbm, incnt_hbm,
                order_hbm, src2_hbm, dst2_hbm, deg_hbm, cnt_hbm, acc_sh, s2_sh,
                d2_sh, map_l, onesbuf, srcbuf, dstbuf, s2buf, d2buf, posbuf,
                fsrcbuf, fdstbuf, cstage, icbuf, dstage, stage):
    del order_hbm
    c = lax.axis_index("c")
    s = lax.axis_index("s")
    w = s * NC + c
    chunk = kpad // NS
    lbase = s * SEG
    pltpu.sync_copy(map_hbm, map_l)
    _zero_shared_1d(s, acc_sh, onesbuf, kpad)
    one = jnp.ones((L,), jnp.float32)
    ktr = jnp.full((L,), k, jnp.int32) + w
    zi = jnp.zeros((L,), jnp.int32)

    def fill(i, _):
        onesbuf[pl.ds(i * L, L)] = one
        fsrcbuf[pl.ds(i * L, L)] = zi
        fdstbuf[pl.ds(i * L, L)] = ktr
        return 0

    lax.fori_loop(0, CE // L, fill, 0)
    pltpu.sync_copy(incnt_hbm.at[pl.ds(w * L, L)], icbuf)
    plsc.subcore_barrier()
    icsc = icbuf[pl.ds(0, L)][0]
    lbv = zi + lbase
    dumpv = zi + (lbase + SEG - 1)

    def chunk_body(i, cnt):
        @pl.when(i * CE < icsc)
        def _():
            base = w * in_stride + i * CE
            pltpu.sync_copy(src_hbm.at[pl.ds(base, CE)], srcbuf)
            pltpu.sync_copy(dst_hbm.at[pl.ds(base, CE)], dstbuf)
            ccnt = zi
            for g2 in range(CE // L):
                sidx = srcbuf[pl.ds(g2 * L, L)]
                didx = dstbuf[pl.ds(g2 * L, L)]
                ms = plsc.load_gather(map_l, [sidx])
                md = plsc.load_gather(map_l, [didx])
                valid = (ms >= 0) & (md >= 0)
                vi = valid.astype(jnp.int32)
                excl = plsc.cumsum(vi) - vi
                pos = lbv + cnt + ccnt + excl
                posbuf[pl.ds(g2 * L, L)] = jnp.where(valid, pos, dumpv)
                s2buf[pl.ds(g2 * L, L)] = jnp.maximum(ms, zi)
                d2buf[pl.ds(g2 * L, L)] = jnp.where(valid, md, ktr)
                ccnt = ccnt + plsc.all_reduce_population_count(valid)
            cstage[pl.ds(0, L)] = ccnt
            pltpu.sync_copy(s2buf, s2_sh.at[posbuf])
            pltpu.sync_copy(d2buf, d2_sh.at[posbuf])
            pltpu.sync_copy(onesbuf, acc_sh.at[d2buf], add=True)

        cc = cstage[pl.ds(0, L)]
        pv = (zi + icsc) > (zi + i * CE)
        return cnt + jnp.where(pv, cc, zi)

    cstage[pl.ds(0, L)] = zi
    cnt = lax.fori_loop(0, NCHUNK, chunk_body, zi)
    cstage[pl.ds(0, L)] = cnt
    iota = lax.iota(jnp.int32, L)
    for i in range(CE // L):
        posbuf[pl.ds(i * L, L)] = lbv + cnt + iota + (i * L)
    pltpu.sync_copy(fsrcbuf, s2_sh.at[posbuf])
    pltpu.sync_copy(fdstbuf, d2_sh.at[posbuf])
    pltpu.sync_copy(cstage, cnt_hbm.at[pl.ds(w * L, L)])
    csc = cstage[pl.ds(0, L)][0]

    def dchunk(i, _):
        @pl.when(i * DC < csc + CE)
        def _():
            pltpu.sync_copy(s2_sh.at[pl.ds(lbase + i * DC, DC)], dstage)
            pltpu.sync_copy(dstage, src2_hbm.at[pl.ds(w * SEG + i * DC, DC)])
            pltpu.sync_copy(d2_sh.at[pl.ds(lbase + i * DC, DC)], dstage)
            pltpu.sync_copy(dstage, dst2_hbm.at[pl.ds(w * SEG + i * DC, DC)])

        return 0

    lax.fori_loop(0, SEG // DC, dchunk, 0)
    plsc.subcore_barrier()
    pltpu.sync_copy(acc_sh.at[pl.ds(s * chunk, chunk)], stage)
    pltpu.sync_copy(stage, deg_hbm.at[pl.ds(c * kpad + s * chunk, chunk)])


def _remap(npad, in_stride, k, kpad, mapping, src, dst, incnt, order):
    return pl.kernel(
        functools.partial(_remap_body, npad, in_stride, k, kpad),
        out_type=(
            jax.ShapeDtypeStruct((NSEG,), jnp.int32),
            jax.ShapeDtypeStruct((NSEG,), jnp.int32),
            jax.ShapeDtypeStruct((NC * kpad,), jnp.float32),
            jax.ShapeDtypeStruct((NW * L,), jnp.int32),
        ),
        mesh=_mesh,
        compiler_params=_SC_PARAMS,
        scratch_types=[
            pltpu.VMEM_SHARED((kpad,), jnp.float32),
            pltpu.VMEM_SHARED((NS * SEG,), jnp.int32),
            pltpu.VMEM_SHARED((NS * SEG,), jnp.int32),
            pltpu.VMEM((npad,), jnp.int32),
            pltpu.VMEM((CE,), jnp.float32),
            pltpu.VMEM((CE,), jnp.int32),
            pltpu.VMEM((CE,), jnp.int32),
            pltpu.VMEM((CE,), jnp.int32),
            pltpu.VMEM((CE,), jnp.int32),
            pltpu.VMEM((CE,), jnp.int32),
            pltpu.VMEM((CE,), jnp.int32),
            pltpu.VMEM((CE,), jnp.int32),
            pltpu.VMEM((L,), jnp.int32),
            pltpu.VMEM((L,), jnp.int32),
            pltpu.VMEM((DC,), jnp.int32),
            pltpu.VMEM((kpad // NS,), jnp.float32),
        ],
    )(mapping, src, dst, incnt, order)


def _tc1_body(x_ref, w_ref, deg_ref, h1_ref, g_ref, dinv_ref):
    deg = deg_ref[...]
    dinv = lax.rsqrt(deg[0] + deg[1] + 1.0)
    h1 = jnp.dot(x_ref[...], w_ref[...], preferred_element_type=jnp.float32)
    h1_ref[...] = h1
    g_ref[...] = dinv * h1
    dinv_ref[...] = dinv


def _tc1(x_p, w, degacc3):
    npad = x_p.shape[0]
    return pl.pallas_call(
        _tc1_body,
        out_shape=(
            jax.ShapeDtypeStruct((npad, H), jnp.float32),
            jax.ShapeDtypeStruct((npad, H), jnp.float32),
            jax.ShapeDtypeStruct((npad, 1), jnp.float32),
        ),
    )(x_p, w, degacc3)


def _tc2_body(S_ref, h1_ref, dinv_ref, b_ref, wp_ref, bp_ref, h_ref, sb_ref,
              gp_ref):
    S = S_ref[...]
    dinv = dinv_ref[...]
    h1 = h1_ref[...]
    A = dinv * (S[0] + S[1]) + (dinv * dinv) * h1 + b_ref[...]
    h = jnp.maximum(A, 0.0)
    h_ref[...] = h
    hp = jnp.dot(h, wp_ref[...], preferred_element_type=jnp.float32)
    sb_ref[...] = (dinv * dinv) * hp + bp_ref[...]
    gp_ref[...] = dinv * hp


def _tc2(S, h1, dinv, b2d, wp, bp2d):
    npad = h1.shape[0]
    return pl.pallas_call(
        _tc2_body,
        out_shape=(
            jax.ShapeDtypeStruct((npad, H), jnp.float32),
            jax.ShapeDtypeStruct((npad, 1), jnp.float32),
            jax.ShapeDtypeStruct((npad, 1), jnp.float32),
        ),
    )(S, h1, dinv, b2d, wp, bp2d)


def _readout_piece(kq, kpad, xn, vals):
    gate = jnp.tanh(vals)
    rowid = lax.broadcasted_iota(jnp.int32, (kpad, 1), 0)
    mask = rowid < kq
    xg = jnp.where(mask, xn * gate, 0.0)
    rmax = jnp.max(jnp.where(mask, xn * gate, -jnp.inf), axis=0,
                   keepdims=True)
    rmean = jnp.sum(xg, axis=0, keepdims=True) / kq
    return xg, jnp.concatenate([rmax, rmean], axis=1)


def _tc3_body(kq, xn_ref, vals_ref, wn_ref, deg_ref, r_ref, h1_ref, g_ref,
              dinv_ref):
    kpad = xn_ref.shape[0]
    xg, r = _readout_piece(kq, kpad, xn_ref[...], vals_ref[...])
    r_ref[...] = r
    deg = deg_ref[...]
    dinv = lax.rsqrt(deg[0] + deg[1] + 1.0)
    h1 = jnp.dot(xg, wn_ref[...], preferred_element_type=jnp.float32)
    h1_ref[...] = h1
    g_ref[...] = dinv * h1
    dinv_ref[...] = dinv


def _tc3(kq, xn, vals2d, wn, degacc3):
    kpad = xn.shape[0]
    return pl.pallas_call(
        functools.partial(_tc3_body, kq),
        out_shape=(
            jax.ShapeDtypeStruct((1, 2 * H), jnp.float32),
            jax.ShapeDtypeStruct((kpad, H), jnp.float32),
            jax.ShapeDtypeStruct((kpad, H), jnp.float32),
            jax.ShapeDtypeStruct((kpad, 1), jnp.float32),
        ),
    )(xn, vals2d, wn, degacc3)


def _readout_body(kq, xn_ref, vals_ref, r_ref):
    kpad = xn_ref.shape[0]
    _, r = _readout_piece(kq, kpad, xn_ref[...], vals_ref[...])
    r_ref[...] = r


def _readout(kq, xn, vals2d):
    return pl.pallas_call(
        functools.partial(_readout_body, kq),
        out_shape=jax.ShapeDtypeStruct((1, 2 * H), jnp.float32),
    )(xn, vals2d)


def _head_body(r1_ref, r2_ref, r3_ref, w1_ref, b1_ref, w2_ref, b2_ref, w3_ref,
               b3_ref, o_ref):
    z = r1_ref[...] + r2_ref[...] + r3_ref[...]
    z = jnp.maximum(
        jnp.dot(z, w1_ref[...], preferred_element_type=jnp.float32)
        + b1_ref[...], 0.0)
    z = jnp.maximum(
        jnp.dot(z, w2_ref[...], preferred_element_type=jnp.float32)
        + b2_ref[...], 0.0)
    z = jnp.dot(z, w3_ref[...], preferred_element_type=jnp.float32) + b3_ref[...]
    m = jnp.max(z, axis=1, keepdims=True)
    o_ref[...] = z - m - jnp.log(
        jnp.sum(jnp.exp(z - m), axis=1, keepdims=True))


def _head(r1, r2, r3, wl1, bl1, wl2, bl2, wl3, bl3):
    return pl.pallas_call(
        _head_body,
        out_shape=jax.ShapeDtypeStruct((1, bl3.shape[1]), jnp.float32),
    )(r1, r2, r3, wl1, bl1, wl2, bl2, wl3, bl3)


def kernel(x, edge_index, batch, W1, b1, Wp1, bp1, W2, b2, Wp2, bp2, W3, b3,
           Wp3, bp3, Wl1, bl1, Wl2, bl2, Wl3, bl3):
    src = edge_index[0].astype(jnp.int32)
    dst = edge_index[1].astype(jnp.int32)
    x_p = jnp.pad(x, ((0, NP1 - N0), (0, 0)))
    cnt_full = jnp.full((NW * L,), PERW, jnp.int32)

    degacc = _deg(NP1, dst)
    h11, g1, dinv1 = _tc1(x_p, W1, degacc.reshape(NC, NP1, 1))
    S1 = _agg(NP1, PERW, g1, src, dst, cnt_full)
    h1o, sb1, gp1 = _tc2(S1, h11, dinv1, b1.reshape(1, H), Wp1,
                         bp1.reshape(1, 1))
    sacc1 = _sagg(NP1, PERW, gp1.reshape(NP1), src, dst, cnt_full)
    perm1, vals1, map1 = _topk(NP1, N0, N0 // 2, NP2, sacc1,
                               dinv1.reshape(NP1), sb1.reshape(NP1))
    xn1 = _gatherrows(NP2, h1o, perm1)
    src2, dst2, deg2, cnt2 = _remap(NP1, PERW, N0 // 2, NP2, map1, src, dst,
                                    cnt_full, xn1)
    r1, h12, g2, dinv2 = _tc3(N0 // 2, xn1, vals1.reshape(NP2, 1), W2,
                              deg2.reshape(NC, NP2, 1))

    S2 = _agg(NP2, SEG, g2, src2, dst2, cnt2)
    h2o, sb2, gp2 = _tc2(S2, h12, dinv2, b2.reshape(1, H), Wp2,
                         bp2.reshape(1, 1))
    sacc2 = _sagg(NP2, SEG, gp2.reshape(NP2), src2, dst2, cnt2)
    perm2, vals2, map2 = _topk(NP2, N0 // 2, N0 // 4, NP3, sacc2,
                               dinv2.reshape(NP2), sb2.reshape(NP2))
    xn2 = _gatherrows(NP3, h2o, perm2)
    src3, dst3, deg3, cnt3 = _remap(NP2, SEG, N0 // 4, NP3, map2, src2, dst2,
                                    cnt2, xn2)
    r2, h13, g3, dinv3 = _tc3(N0 // 4, xn2, vals2.reshape(NP3, 1), W3,
                              deg3.reshape(NC, NP3, 1))

    S3 = _agg(NP3, SEG, g3, src3, dst3, cnt3)
    h3o, sb3, gp3 = _tc2(S3, h13, dinv3, b3.reshape(1, H), Wp3,
                         bp3.reshape(1, 1))
    sacc3 = _sagg(NP3, SEG, gp3.reshape(NP3), src3, dst3, cnt3)
    perm3, vals3, _map3 = _topk(NP3, N0 // 4, N0 // 8, NP4, sacc3,
                                dinv3.reshape(NP3), sb3.reshape(NP3))
    xn3 = _gatherrows(NP4, h3o, perm3)
    r3 = _readout(N0 // 8, xn3, vals3.reshape(NP4, 1))

    return _head(r1, r2, r3, Wl1, bl1.reshape(1, H), Wl2,
                 bl2.reshape(1, H // 2), Wl3, bl3.reshape(1, 10))
